# baseline (speedup 1.0000x reference)
import jax
import jax.numpy as jnp
from jax.experimental import pallas as pl
from jax.experimental.pallas import tpu as pltpu

N_EXPERT = 16
DIM_IN = 2048
TILE = 2048
KC = 1024
NK = DIM_IN // KC


def _gating_kernel(x_ref, wt_ref, cw_ref, ei_ref, acc_ref):
    k = pl.program_id(1)
    partial = jax.lax.dot_general(
        x_ref[...], wt_ref[...], (((1,), (0,)), ((), ())),
        preferred_element_type=jnp.float32,
    )

    @pl.when(k == 0)
    def _init():
        acc_ref[...] = partial

    @pl.when(k > 0)
    def _accum():
        acc_ref[...] += partial

    @pl.when(k == NK - 1)
    def _finish():
        logits = acc_ref[...]
        t = logits.shape[0]
        iota = jax.lax.broadcasted_iota(jnp.int32, (t, N_EXPERT), 1)

        m1 = jnp.max(logits, axis=-1, keepdims=True)
        idx1 = jnp.min(
            jnp.where(logits == m1, iota, N_EXPERT), axis=-1, keepdims=True
        )
        masked = jnp.where(iota == idx1, -jnp.inf, logits)
        m2 = jnp.max(masked, axis=-1, keepdims=True)
        idx2 = jnp.min(
            jnp.where(masked == m2, iota, N_EXPERT), axis=-1, keepdims=True
        )

        z = jnp.sum(jnp.exp(logits - m1), axis=-1, keepdims=True)
        p1 = 1.0 / z
        p2 = jnp.exp(m2 - m1) / z
        den = p1 + p2 + 1e-09
        cw_ref[:, 0:1] = p1 / den
        cw_ref[:, 1:2] = p2 / den
        ei_ref[:, 0:1] = idx1
        ei_ref[:, 1:2] = idx2


def kernel(x, W):
    b, n, d = x.shape
    tokens = b * n
    xf = x.reshape(tokens, d)
    wt = W.T
    grid = (tokens // TILE, NK)
    cw, ei = pl.pallas_call(
        _gating_kernel,
        grid=grid,
        in_specs=[
            pl.BlockSpec((TILE, KC), lambda i, k: (i, k)),
            pl.BlockSpec((KC, N_EXPERT), lambda i, k: (k, 0)),
        ],
        out_specs=[
            pl.BlockSpec((TILE, 2), lambda i, k: (i, 0)),
            pl.BlockSpec((TILE, 2), lambda i, k: (i, 0)),
        ],
        out_shape=[
            jax.ShapeDtypeStruct((tokens, 2), jnp.float32),
            jax.ShapeDtypeStruct((tokens, 2), jnp.int32),
        ],
        scratch_shapes=[pltpu.VMEM((TILE, N_EXPERT), jnp.float32)],
        compiler_params=pltpu.CompilerParams(
            dimension_semantics=("parallel", "arbitrary"),
        ),
    )(xf, wt)
    return cw.reshape(b, n, 2), ei.reshape(b, n, 2)
